# Initial kernel scaffold; baseline (speedup 1.0000x reference)
import jax
import jax.numpy as jnp
from jax.experimental import pallas as pl


def kernel(x, edge_index, W1, b1, W2, b2, Wf1, bf1, Wf2, bf2):
    raise NotImplementedError("write your pallas kernel here")


# speedup vs baseline: 30.5536x; 30.5536x over previous
import functools

import jax
import jax.numpy as jnp
from jax import lax
from jax.experimental import pallas as pl
from jax.experimental.pallas import tpu as pltpu
from jax.experimental.pallas import tpu_sc as plsc

N = 100000
E = 3200000
NC = 2
NS = 16
NW = NC * NS
NPAD = 102400
TSL = NPAD // NS
G = 125
WB = 400

A_GROUPS = E // NW // G
BC_CH = 32
BC_GPC = E // NS // BC_CH // G

_mesh = plsc.VectorSubcoreMesh(core_axis_name="c", subcore_axis_name="s")
_sc_params = pltpu.CompilerParams(use_tc_tiling_on_sc=False)


def _zero_fill(ref, rows):
    def body(i, _):
        ref[i] = jnp.zeros((16,), jnp.float32)
        return 0
    lax.fori_loop(0, rows, body, 0)


def _zero_fill_1d(ref, n16):
    def body(i, _):
        ref[pl.ds(i * 16, 16)] = jnp.zeros((16,), jnp.float32)
        return 0
    lax.fori_loop(0, n16, body, 0)


def _deg_body(dst_hbm, out_hbm, acc, dst_v, ones_v, zb_v):
    c = lax.axis_index("c")
    s = lax.axis_index("s")
    w = c * NS + s
    _zero_fill_1d(zb_v, TSL // 16)
    pltpu.sync_copy(zb_v, acc.at[pl.ds(s * TSL, TSL)])
    def ones_body(i, _):
        ones_v[pl.ds(i * 16, 16)] = jnp.ones((16,), jnp.float32)
        return 0
    lax.fori_loop(0, 8, ones_body, 0)
    plsc.subcore_barrier()
    pltpu.sync_copy(dst_hbm.at[w], dst_v)
    def body(j, _):
        pltpu.sync_copy(ones_v.at[pl.ds(0, G)], acc.at[dst_v.at[j]], add=True)
        return 0
    lax.fori_loop(0, A_GROUPS, body, 0)
    plsc.subcore_barrier()
    pltpu.sync_copy(acc.at[pl.ds(s * TSL, TSL)], zb_v)
    pltpu.sync_copy(zb_v, out_hbm.at[c, pl.ds(s * TSL, TSL)])


@functools.partial(
    pl.kernel,
    out_type=jax.ShapeDtypeStruct((NC, NPAD), jnp.float32),
    mesh=_mesh,
    compiler_params=_sc_params,
    scratch_types=[
        pltpu.VMEM_SHARED((NPAD,), jnp.float32),
        pltpu.VMEM((A_GROUPS, G), jnp.int32),
        pltpu.VMEM((128,), jnp.float32),
        pltpu.VMEM((TSL,), jnp.float32),
    ],
)
def _sc_degree(dst_hbm, out_hbm, acc, dst_v, ones_v, zb_v):
    _deg_body(dst_hbm, out_hbm, acc, dst_v, ones_v, zb_v)


def _p1_body(y1c_hbm, src_hbm, dst_hbm, out_hbm, tbl, acc,
             src_v, dst_v, rows_v, yb_v, zb_v):
    c = lax.axis_index("c")
    s = lax.axis_index("s")
    pltpu.sync_copy(y1c_hbm.at[c, pl.ds(s * TSL, TSL)], yb_v)
    pltpu.sync_copy(yb_v, tbl.at[pl.ds(s * TSL, TSL)])
    _zero_fill_1d(zb_v, TSL // 16)
    pltpu.sync_copy(zb_v, acc.at[pl.ds(s * TSL, TSL)])
    plsc.subcore_barrier()
    def chunk(k, _):
        pltpu.sync_copy(src_hbm.at[s, k], src_v)
        pltpu.sync_copy(dst_hbm.at[s, k], dst_v)
        def body(j, _):
            pltpu.sync_copy(tbl.at[src_v.at[j]], rows_v)
            pltpu.sync_copy(rows_v, acc.at[dst_v.at[j]], add=True)
            return 0
        lax.fori_loop(0, BC_GPC, body, 0)
        return 0
    lax.fori_loop(0, BC_CH, chunk, 0)
    plsc.subcore_barrier()
    pltpu.sync_copy(acc.at[pl.ds(s * TSL, TSL)], zb_v)
    pltpu.sync_copy(zb_v, out_hbm.at[c, pl.ds(s * TSL, TSL)])


@functools.partial(
    pl.kernel,
    out_type=jax.ShapeDtypeStruct((NC, NPAD), jnp.float32),
    mesh=_mesh,
    compiler_params=_sc_params,
    scratch_types=[
        pltpu.VMEM_SHARED((NPAD,), jnp.float32),
        pltpu.VMEM_SHARED((NPAD,), jnp.float32),
        pltpu.VMEM((BC_GPC, G), jnp.int32),
        pltpu.VMEM((BC_GPC, G), jnp.int32),
        pltpu.VMEM((G,), jnp.float32),
        pltpu.VMEM((TSL,), jnp.float32),
        pltpu.VMEM((TSL,), jnp.float32),
    ],
)
def _sc_prop1(y1c_hbm, src_hbm, dst_hbm, out_hbm, tbl, acc,
              src_v, dst_v, rows_v, yb_v, zb_v):
    _p1_body(y1c_hbm, src_hbm, dst_hbm, out_hbm, tbl, acc,
             src_v, dst_v, rows_v, yb_v, zb_v)


def _p2_body(y2t_hbm, src_hbm, dst_hbm, out_hbm, acc,
             src_v, dst_v, rows_v, zb_v):
    c = lax.axis_index("c")
    s = lax.axis_index("s")
    tbl = y2t_hbm.at[c]
    _zero_fill(zb_v, WB)
    def zinit(t, _):
        pltpu.sync_copy(zb_v, acc.at[pl.ds(s * TSL + t * WB, WB)])
        return 0
    lax.fori_loop(0, TSL // WB, zinit, 0)
    plsc.subcore_barrier()
    def chunk(k, _):
        pltpu.sync_copy(src_hbm.at[s, k], src_v)
        pltpu.sync_copy(dst_hbm.at[s, k], dst_v)
        def body(j, _):
            pltpu.sync_copy(tbl.at[src_v.at[j]], rows_v)
            pltpu.sync_copy(rows_v, acc.at[dst_v.at[j]], add=True)
            return 0
        lax.fori_loop(0, BC_GPC, body, 0)
        return 0
    lax.fori_loop(0, BC_CH, chunk, 0)
    plsc.subcore_barrier()
    def wback(t, _):
        pltpu.sync_copy(acc.at[pl.ds(s * TSL + t * WB, WB)], zb_v)
        pltpu.sync_copy(zb_v, out_hbm.at[c, pl.ds(s * TSL + t * WB, WB)])
        return 0
    lax.fori_loop(0, TSL // WB, wback, 0)


@functools.partial(
    pl.kernel,
    out_type=jax.ShapeDtypeStruct((NC, NPAD, 16), jnp.float32),
    mesh=_mesh,
    compiler_params=_sc_params,
    scratch_types=[
        pltpu.VMEM_SHARED((NPAD, 16), jnp.float32),
        pltpu.VMEM((BC_GPC, G), jnp.int32),
        pltpu.VMEM((BC_GPC, G), jnp.int32),
        pltpu.VMEM((G, 16), jnp.float32),
        pltpu.VMEM((WB, 16), jnp.float32),
    ],
)
def _sc_prop16(y2t_hbm, src_hbm, dst_hbm, out_hbm, acc,
               src_v, dst_v, rows_v, zb_v):
    _p2_body(y2t_hbm, src_hbm, dst_hbm, out_hbm, acc,
             src_v, dst_v, rows_v, zb_v)


BLK = 6400
GRID = NPAD // BLK


def _tc_prep_body(deg2_ref, x_ref, dinv_ref, y1c_ref):
    d = deg2_ref[0, :] + deg2_ref[1, :] + 1.0
    dv = lax.rsqrt(d)
    dinv_ref[0, :] = dv
    y1c_ref[0, :] = x_ref[:, 0] * dv
    y1c_ref[1, :] = x_ref[:, 1] * dv


def _tc_prep(deg2, xp):
    return pl.pallas_call(
        _tc_prep_body,
        grid=(GRID,),
        in_specs=[
            pl.BlockSpec((NC, BLK), lambda i: (0, i)),
            pl.BlockSpec((BLK, 2), lambda i: (i, 0)),
        ],
        out_specs=[
            pl.BlockSpec((1, BLK), lambda i: (0, i)),
            pl.BlockSpec((NC, BLK), lambda i: (0, i)),
        ],
        out_shape=[
            jax.ShapeDtypeStruct((1, NPAD), jnp.float32),
            jax.ShapeDtypeStruct((NC, NPAD), jnp.float32),
        ],
    )(deg2, xp)


def _tc_mid_body(p1c_ref, y1c_ref, dinv_ref, W1_ref, b1_ref, y2t_ref):
    dv = dinv_ref[0, :]
    prop0 = dv * (p1c_ref[0, :] + y1c_ref[0, :])
    prop1 = dv * (p1c_ref[1, :] + y1c_ref[1, :])
    h = (prop0[:, None] * W1_ref[0:1, :]
         + prop1[:, None] * W1_ref[1:2, :]
         + b1_ref[...])
    h = jnp.maximum(h, 0.0)
    y2 = h * dv[:, None]
    y2t_ref[0] = y2[:, :16]
    y2t_ref[1] = y2[:, 16:]


def _tc_mid(p1c, y1c, dinv, W1, b1):
    return pl.pallas_call(
        _tc_mid_body,
        grid=(GRID,),
        in_specs=[
            pl.BlockSpec((NC, BLK), lambda i: (0, i)),
            pl.BlockSpec((NC, BLK), lambda i: (0, i)),
            pl.BlockSpec((1, BLK), lambda i: (0, i)),
            pl.BlockSpec((2, 32), lambda i: (0, 0)),
            pl.BlockSpec((1, 32), lambda i: (0, 0)),
        ],
        out_specs=pl.BlockSpec((NC, BLK, 16), lambda i: (0, i, 0)),
        out_shape=jax.ShapeDtypeStruct((NC, NPAD, 16), jnp.float32),
    )(p1c, y1c, dinv, W1, b1.reshape(1, 32))


def _tc_final_body(p2_ref, y2t_ref, dinv_ref, W2_ref, b2_ref,
                   Wf1_ref, bf1_ref, Wf2_ref, bf2_ref, out_ref):
    dv = dinv_ref[0, :][:, None]
    y2full = jnp.concatenate([y2t_ref[0], y2t_ref[1]], axis=1)
    p2full = jnp.concatenate([p2_ref[0], p2_ref[1]], axis=1)
    prop2 = dv * (p2full + y2full)
    h2 = jnp.maximum(
        jnp.dot(prop2, W2_ref[...], preferred_element_type=jnp.float32)
        + b2_ref[...], 0.0)
    h3 = jnp.maximum(
        jnp.dot(h2, Wf1_ref[...], preferred_element_type=jnp.float32)
        + bf1_ref[...], 0.0)
    out_ref[...] = (
        jnp.dot(h3, Wf2_ref[...], preferred_element_type=jnp.float32)
        + bf2_ref[...])


def _tc_final(p2, y2t, dinv, W2, b2, Wf1, bf1, Wf2, bf2):
    return pl.pallas_call(
        _tc_final_body,
        grid=(GRID,),
        in_specs=[
            pl.BlockSpec((NC, BLK, 16), lambda i: (0, i, 0)),
            pl.BlockSpec((NC, BLK, 16), lambda i: (0, i, 0)),
            pl.BlockSpec((1, BLK), lambda i: (0, i)),
            pl.BlockSpec((32, 32), lambda i: (0, 0)),
            pl.BlockSpec((1, 32), lambda i: (0, 0)),
            pl.BlockSpec((32, 32), lambda i: (0, 0)),
            pl.BlockSpec((1, 32), lambda i: (0, 0)),
            pl.BlockSpec((32, 1), lambda i: (0, 0)),
            pl.BlockSpec((1, 1), lambda i: (0, 0)),
        ],
        out_specs=pl.BlockSpec((BLK, 1), lambda i: (i, 0)),
        out_shape=jax.ShapeDtypeStruct((NPAD, 1), jnp.float32),
    )(p2, y2t, dinv, W2, b2.reshape(1, 32), Wf1, bf1.reshape(1, 32),
      Wf2, bf2.reshape(1, 1))


def kernel(x, edge_index, W1, b1, W2, b2, Wf1, bf1, Wf2, bf2):
    src = edge_index[0].astype(jnp.int32)
    dst = edge_index[1].astype(jnp.int32)
    dstA = dst.reshape(NW, A_GROUPS, G)
    srcBC = src.reshape(NS, BC_CH, BC_GPC, G)
    dstBC = dst.reshape(NS, BC_CH, BC_GPC, G)
    xp = jnp.pad(x, ((0, NPAD - N), (0, 0)))

    deg2 = _sc_degree(dstA)
    dinv, y1c = _tc_prep(deg2, xp)
    p1c = _sc_prop1(y1c, srcBC, dstBC)
    y2t = _tc_mid(p1c, y1c, dinv, W1, b1)
    p2 = _sc_prop16(y2t, srcBC, dstBC)
    outp = _tc_final(p2, y2t, dinv, W2, b2, Wf1, bf1, Wf2, bf2)
    return outp[:N]


# speedup vs baseline: 55.0539x; 1.8019x over previous
import functools

import jax
import jax.numpy as jnp
from jax import lax
from jax.experimental import pallas as pl
from jax.experimental.pallas import tpu as pltpu
from jax.experimental.pallas import tpu_sc as plsc

N = 100000
E = 3200000
NC = 2
NS = 16
NW = NC * NS
NPAD = 102400
TSL = NPAD // NS
G = 125
WB = 200
D = 5

A_GROUPS = E // NW // G
BC_CH = 32
BC_GPC = E // NS // BC_CH // G

_mesh = plsc.VectorSubcoreMesh(core_axis_name="c", subcore_axis_name="s")
_sc_params = pltpu.CompilerParams(use_tc_tiling_on_sc=False)


def _zero_fill(ref, rows):
    def body(i, _):
        ref[i] = jnp.zeros((16,), jnp.float32)
        return 0
    lax.fori_loop(0, rows, body, 0)


def _zero_fill_1d(ref, n16):
    def body(i, _):
        ref[pl.ds(i * 16, 16)] = jnp.zeros((16,), jnp.float32)
        return 0
    lax.fori_loop(0, n16, body, 0)


def _deg_body(dst_hbm, out_hbm, acc, dst_v, ones_v, zb_v):
    c = lax.axis_index("c")
    s = lax.axis_index("s")
    w = c * NS + s
    _zero_fill_1d(zb_v, TSL // 16)
    pltpu.sync_copy(zb_v, acc.at[pl.ds(s * TSL, TSL)])
    def ones_body(i, _):
        ones_v[pl.ds(i * 16, 16)] = jnp.ones((16,), jnp.float32)
        return 0
    lax.fori_loop(0, 8, ones_body, 0)
    plsc.subcore_barrier()
    pltpu.sync_copy(dst_hbm.at[w], dst_v)
    def body(j, _):
        pltpu.sync_copy(ones_v.at[pl.ds(0, G)], acc.at[dst_v.at[j]], add=True)
        return 0
    lax.fori_loop(0, A_GROUPS, body, 0)
    plsc.subcore_barrier()
    pltpu.sync_copy(acc.at[pl.ds(s * TSL, TSL)], zb_v)
    pltpu.sync_copy(zb_v, out_hbm.at[c, pl.ds(s * TSL, TSL)])


@functools.partial(
    pl.kernel,
    out_type=jax.ShapeDtypeStruct((NC, NPAD), jnp.float32),
    mesh=_mesh,
    compiler_params=_sc_params,
    scratch_types=[
        pltpu.VMEM_SHARED((NPAD,), jnp.float32),
        pltpu.VMEM((A_GROUPS, G), jnp.int32),
        pltpu.VMEM((128,), jnp.float32),
        pltpu.VMEM((TSL,), jnp.float32),
    ],
)
def _sc_degree(dst_hbm, out_hbm, acc, dst_v, ones_v, zb_v):
    _deg_body(dst_hbm, out_hbm, acc, dst_v, ones_v, zb_v)


def _p1_body(y1c_hbm, src_hbm, dst_hbm, out_hbm, tbl, acc,
             src_v, dst_v, rows_v, yb_v, zb_v, gsem, ssem):
    c = lax.axis_index("c")
    s = lax.axis_index("s")
    pltpu.sync_copy(y1c_hbm.at[c, pl.ds(s * TSL, TSL)], yb_v)
    pltpu.sync_copy(yb_v, tbl.at[pl.ds(s * TSL, TSL)])
    _zero_fill_1d(zb_v, TSL // 16)
    pltpu.sync_copy(zb_v, acc.at[pl.ds(s * TSL, TSL)])
    plsc.subcore_barrier()
    def chunk(k, _):
        pltpu.sync_copy(src_hbm.at[s, k], src_v)
        pltpu.sync_copy(dst_hbm.at[s, k], dst_v)
        def quint(q, _):
            gs = [pltpu.async_copy(tbl.at[src_v.at[q * D + d]],
                                   rows_v.at[d], gsem.at[d])
                  for d in range(D)]
            ss = []
            for d in range(D):
                gs[d].wait()
                ss.append(pltpu.async_copy(rows_v.at[d],
                                           acc.at[dst_v.at[q * D + d]],
                                           ssem.at[d], add=True))
            for d in range(D):
                ss[d].wait()
            return 0
        lax.fori_loop(0, BC_GPC // D, quint, 0)
        return 0
    lax.fori_loop(0, BC_CH, chunk, 0)
    plsc.subcore_barrier()
    pltpu.sync_copy(acc.at[pl.ds(s * TSL, TSL)], zb_v)
    pltpu.sync_copy(zb_v, out_hbm.at[c, pl.ds(s * TSL, TSL)])


@functools.partial(
    pl.kernel,
    out_type=jax.ShapeDtypeStruct((NC, NPAD), jnp.float32),
    mesh=_mesh,
    compiler_params=_sc_params,
    scratch_types=[
        pltpu.VMEM_SHARED((NPAD,), jnp.float32),
        pltpu.VMEM_SHARED((NPAD,), jnp.float32),
        pltpu.VMEM((BC_GPC, G), jnp.int32),
        pltpu.VMEM((BC_GPC, G), jnp.int32),
        pltpu.VMEM((D, G), jnp.float32),
        pltpu.VMEM((TSL,), jnp.float32),
        pltpu.VMEM((TSL,), jnp.float32),
        pltpu.SemaphoreType.DMA((D,)),
        pltpu.SemaphoreType.DMA((D,)),
    ],
)
def _sc_prop1(y1c_hbm, src_hbm, dst_hbm, out_hbm, tbl, acc,
              src_v, dst_v, rows_v, yb_v, zb_v, gsem, ssem):
    _p1_body(y1c_hbm, src_hbm, dst_hbm, out_hbm, tbl, acc,
             src_v, dst_v, rows_v, yb_v, zb_v, gsem, ssem)


def _p2_body(y2t_hbm, src_hbm, dst_hbm, out_hbm, acc,
             src_v, dst_v, rows_v, zb_v, gsem, ssem):
    c = lax.axis_index("c")
    s = lax.axis_index("s")
    tbl = y2t_hbm.at[c]
    _zero_fill(zb_v, WB)
    def zinit(t, _):
        pltpu.sync_copy(zb_v, acc.at[pl.ds(s * TSL + t * WB, WB)])
        return 0
    lax.fori_loop(0, TSL // WB, zinit, 0)
    plsc.subcore_barrier()
    def chunk(k, _):
        pltpu.sync_copy(src_hbm.at[s, k], src_v)
        pltpu.sync_copy(dst_hbm.at[s, k], dst_v)
        def quint(q, _):
            gs = [pltpu.async_copy(tbl.at[src_v.at[q * D + d]],
                                   rows_v.at[d], gsem.at[d])
                  for d in range(D)]
            ss = []
            for d in range(D):
                gs[d].wait()
                ss.append(pltpu.async_copy(rows_v.at[d],
                                           acc.at[dst_v.at[q * D + d]],
                                           ssem.at[d], add=True))
            for d in range(D):
                ss[d].wait()
            return 0
        lax.fori_loop(0, BC_GPC // D, quint, 0)
        return 0
    lax.fori_loop(0, BC_CH, chunk, 0)
    plsc.subcore_barrier()
    def wback(t, _):
        pltpu.sync_copy(acc.at[pl.ds(s * TSL + t * WB, WB)], zb_v)
        pltpu.sync_copy(zb_v, out_hbm.at[c, pl.ds(s * TSL + t * WB, WB)])
        return 0
    lax.fori_loop(0, TSL // WB, wback, 0)


@functools.partial(
    pl.kernel,
    out_type=jax.ShapeDtypeStruct((NC, NPAD, 16), jnp.float32),
    mesh=_mesh,
    compiler_params=_sc_params,
    scratch_types=[
        pltpu.VMEM_SHARED((NPAD, 16), jnp.float32),
        pltpu.VMEM((BC_GPC, G), jnp.int32),
        pltpu.VMEM((BC_GPC, G), jnp.int32),
        pltpu.VMEM((D, G, 16), jnp.float32),
        pltpu.VMEM((WB, 16), jnp.float32),
        pltpu.SemaphoreType.DMA((D,)),
        pltpu.SemaphoreType.DMA((D,)),
    ],
)
def _sc_prop16(y2t_hbm, src_hbm, dst_hbm, out_hbm, acc,
               src_v, dst_v, rows_v, zb_v, gsem, ssem):
    _p2_body(y2t_hbm, src_hbm, dst_hbm, out_hbm, acc,
             src_v, dst_v, rows_v, zb_v, gsem, ssem)


BLK = 6400
GRID = NPAD // BLK


def _tc_prep_body(deg2_ref, x_ref, dinv_ref, y1c_ref):
    d = deg2_ref[0, :] + deg2_ref[1, :] + 1.0
    dv = lax.rsqrt(d)
    dinv_ref[0, :] = dv
    y1c_ref[0, :] = x_ref[:, 0] * dv
    y1c_ref[1, :] = x_ref[:, 1] * dv


def _tc_prep(deg2, xp):
    return pl.pallas_call(
        _tc_prep_body,
        grid=(GRID,),
        in_specs=[
            pl.BlockSpec((NC, BLK), lambda i: (0, i)),
            pl.BlockSpec((BLK, 2), lambda i: (i, 0)),
        ],
        out_specs=[
            pl.BlockSpec((1, BLK), lambda i: (0, i)),
            pl.BlockSpec((NC, BLK), lambda i: (0, i)),
        ],
        out_shape=[
            jax.ShapeDtypeStruct((1, NPAD), jnp.float32),
            jax.ShapeDtypeStruct((NC, NPAD), jnp.float32),
        ],
    )(deg2, xp)


def _tc_mid_body(p1c_ref, y1c_ref, dinv_ref, W1_ref, b1_ref, y2t_ref):
    dv = dinv_ref[0, :]
    prop0 = dv * (p1c_ref[0, :] + y1c_ref[0, :])
    prop1 = dv * (p1c_ref[1, :] + y1c_ref[1, :])
    h = (prop0[:, None] * W1_ref[0:1, :]
         + prop1[:, None] * W1_ref[1:2, :]
         + b1_ref[...])
    h = jnp.maximum(h, 0.0)
    y2 = h * dv[:, None]
    y2t_ref[0] = y2[:, :16]
    y2t_ref[1] = y2[:, 16:]


def _tc_mid(p1c, y1c, dinv, W1, b1):
    return pl.pallas_call(
        _tc_mid_body,
        grid=(GRID,),
        in_specs=[
            pl.BlockSpec((NC, BLK), lambda i: (0, i)),
            pl.BlockSpec((NC, BLK), lambda i: (0, i)),
            pl.BlockSpec((1, BLK), lambda i: (0, i)),
            pl.BlockSpec((2, 32), lambda i: (0, 0)),
            pl.BlockSpec((1, 32), lambda i: (0, 0)),
        ],
        out_specs=pl.BlockSpec((NC, BLK, 16), lambda i: (0, i, 0)),
        out_shape=jax.ShapeDtypeStruct((NC, NPAD, 16), jnp.float32),
    )(p1c, y1c, dinv, W1, b1.reshape(1, 32))


def _tc_final_body(p2_ref, y2t_ref, dinv_ref, W2_ref, b2_ref,
                   Wf1_ref, bf1_ref, Wf2_ref, bf2_ref, out_ref):
    dv = dinv_ref[0, :][:, None]
    y2full = jnp.concatenate([y2t_ref[0], y2t_ref[1]], axis=1)
    p2full = jnp.concatenate([p2_ref[0], p2_ref[1]], axis=1)
    prop2 = dv * (p2full + y2full)
    h2 = jnp.maximum(
        jnp.dot(prop2, W2_ref[...], preferred_element_type=jnp.float32)
        + b2_ref[...], 0.0)
    h3 = jnp.maximum(
        jnp.dot(h2, Wf1_ref[...], preferred_element_type=jnp.float32)
        + bf1_ref[...], 0.0)
    out_ref[...] = (
        jnp.dot(h3, Wf2_ref[...], preferred_element_type=jnp.float32)
        + bf2_ref[...])


def _tc_final(p2, y2t, dinv, W2, b2, Wf1, bf1, Wf2, bf2):
    return pl.pallas_call(
        _tc_final_body,
        grid=(GRID,),
        in_specs=[
            pl.BlockSpec((NC, BLK, 16), lambda i: (0, i, 0)),
            pl.BlockSpec((NC, BLK, 16), lambda i: (0, i, 0)),
            pl.BlockSpec((1, BLK), lambda i: (0, i)),
            pl.BlockSpec((32, 32), lambda i: (0, 0)),
            pl.BlockSpec((1, 32), lambda i: (0, 0)),
            pl.BlockSpec((32, 32), lambda i: (0, 0)),
            pl.BlockSpec((1, 32), lambda i: (0, 0)),
            pl.BlockSpec((32, 1), lambda i: (0, 0)),
            pl.BlockSpec((1, 1), lambda i: (0, 0)),
        ],
        out_specs=pl.BlockSpec((BLK, 1), lambda i: (i, 0)),
        out_shape=jax.ShapeDtypeStruct((NPAD, 1), jnp.float32),
    )(p2, y2t, dinv, W2, b2.reshape(1, 32), Wf1, bf1.reshape(1, 32),
      Wf2, bf2.reshape(1, 1))


def kernel(x, edge_index, W1, b1, W2, b2, Wf1, bf1, Wf2, bf2):
    src = edge_index[0].astype(jnp.int32)
    dst = edge_index[1].astype(jnp.int32)
    dstA = dst.reshape(NW, A_GROUPS, G)
    srcBC = src.reshape(NS, BC_CH, BC_GPC, G)
    dstBC = dst.reshape(NS, BC_CH, BC_GPC, G)
    xp = jnp.pad(x, ((0, NPAD - N), (0, 0)))

    deg2 = _sc_degree(dstA)
    dinv, y1c = _tc_prep(deg2, xp)
    p1c = _sc_prop1(y1c, srcBC, dstBC)
    y2t = _tc_mid(p1c, y1c, dinv, W1, b1)
    p2 = _sc_prop16(y2t, srcBC, dstBC)
    outp = _tc_final(p2, y2t, dinv, W2, b2, Wf1, bf1, Wf2, bf2)
    return outp[:N]
